# baseline (speedup 1.0000x reference)
import functools

import jax
import jax.numpy as jnp
from jax import lax
from jax.experimental import pallas as pl
from jax.experimental.pallas import tpu as pltpu
from jax.experimental.pallas import tpu_sc as plsc

_LAMDA = 0.5
_NC = 2
_NS = 16
_NW = _NC * _NS
_L = 16
_IDXROW = 128
_CHUNK = 32


@functools.partial(jax.jit, static_argnames=("batch", "dim"))
def _partial_sums(centers, target_i32, embedding, *, batch, dim):
  rows_per_w = batch // _NW
  idx_rows = rows_per_w // _IDXROW
  groups = dim // _L
  n_chunks = rows_per_w // _CHUNK
  target_2d = target_i32.reshape(batch // _IDXROW, _IDXROW)

  mesh = plsc.VectorSubcoreMesh(
      core_axis_name="c", subcore_axis_name="s", num_cores=_NC, num_subcores=_NS
  )

  @functools.partial(
      pl.kernel,
      out_type=jax.ShapeDtypeStruct((_NW, _L), jnp.float32),
      mesh=mesh,
      scratch_types=[
          pltpu.VMEM((idx_rows, _IDXROW), jnp.int32),
          pltpu.VMEM((2, _CHUNK, dim), jnp.float32),
          pltpu.VMEM((2, _CHUNK, dim), jnp.float32),
          pltpu.VMEM((_L,), jnp.float32),
          pltpu.SemaphoreType.DMA,
          pltpu.SemaphoreType.DMA,
          pltpu.SemaphoreType.DMA,
          pltpu.SemaphoreType.DMA,
      ],
  )
  def k(centers_hbm, tgt_hbm, emb_hbm, out_hbm, idx_v, rows_v, emb_v, acc_v,
        gsem0, gsem1, esem0, esem1):
    wid = lax.axis_index("s") * _NC + lax.axis_index("c")
    base = wid * rows_per_w
    gsems = (gsem0, gsem1)
    esems = (esem0, esem1)

    def issue(kk, slot):
      row = kk // 4
      col = (kk % 4) * _CHUNK
      g = pltpu.async_copy(centers_hbm.at[idx_v.at[row, pl.ds(col, _CHUNK)]],
                           rows_v.at[slot], gsems[slot])
      e = pltpu.async_copy(emb_hbm.at[pl.ds(base + kk * _CHUNK, _CHUNK)],
                           emb_v.at[slot], esems[slot])
      return (g, e)

    pltpu.sync_copy(tgt_hbm.at[pl.ds(wid * idx_rows, idx_rows)], idx_v)
    for b in range(2):
      issue(b, b)

    accs = tuple(jnp.zeros((_L,), jnp.float32) for _ in range(groups))

    def round_body(g, accs):
      for b in range(2):
        kk = 2 * g + b
        row = kk // 4
        col = (kk % 4) * _CHUNK
        pltpu.make_async_copy(centers_hbm.at[idx_v.at[row, pl.ds(col, _CHUNK)]],
                              rows_v.at[b], gsems[b]).wait()
        pltpu.make_async_copy(emb_hbm.at[pl.ds(base + kk * _CHUNK, _CHUNK)],
                              emb_v.at[b], esems[b]).wait()

        @pl.when(g < n_chunks // 2 - 1)
        def _():
          issue(kk + 2, b)

        slot = b

        def row_body(r, accs):
          new = []
          for j in range(groups):
            d = emb_v[slot, r, pl.ds(j * _L, _L)] - rows_v[slot, r, pl.ds(j * _L, _L)]
            new.append(accs[j] + d * d)
          return tuple(new)

        accs = plsc.parallel_loop(0, _CHUNK, carry=accs, unroll=2)(row_body)
      return accs

    accs = lax.fori_loop(0, n_chunks // 2, round_body, accs)

    total = accs[0]
    for j in range(1, groups):
      total = total + accs[j]
    acc_v[...] = total
    pltpu.sync_copy(acc_v, out_hbm.at[wid])

  return k(centers, target_2d, embedding)


def kernel(embedding, target, centers):
  batch, dim = embedding.shape
  partials = _partial_sums(
      centers, target.astype(jnp.int32), embedding, batch=batch, dim=dim
  )
  dist = jnp.sqrt(jnp.sum(partials))
  return (_LAMDA * 0.5 / batch) * dist


# speedup vs baseline: 1.0389x; 1.0389x over previous
import functools

import jax
import jax.numpy as jnp
from jax import lax
from jax.experimental import pallas as pl
from jax.experimental.pallas import tpu as pltpu
from jax.experimental.pallas import tpu_sc as plsc

_LAMDA = 0.5
_NC = 2
_NS = 16
_NW = _NC * _NS
_L = 16
_IDXROW = 128
_CHUNK = 64


@functools.partial(jax.jit, static_argnames=("batch", "dim"))
def _partial_sums(centers, target_i32, embedding, *, batch, dim):
  rows_per_w = batch // _NW
  idx_rows = rows_per_w // _IDXROW
  groups = dim // _L
  n_chunks = rows_per_w // _CHUNK
  target_2d = target_i32.reshape(batch // _IDXROW, _IDXROW)

  mesh = plsc.VectorSubcoreMesh(
      core_axis_name="c", subcore_axis_name="s", num_cores=_NC, num_subcores=_NS
  )

  @functools.partial(
      pl.kernel,
      out_type=jax.ShapeDtypeStruct((_NW, _L), jnp.float32),
      mesh=mesh,
      scratch_types=[
          pltpu.VMEM((idx_rows, _IDXROW), jnp.int32),
          pltpu.VMEM((2, _CHUNK, dim), jnp.float32),
          pltpu.VMEM((2, _CHUNK, dim), jnp.float32),
          pltpu.VMEM((_L,), jnp.float32),
          pltpu.SemaphoreType.DMA,
          pltpu.SemaphoreType.DMA,
          pltpu.SemaphoreType.DMA,
          pltpu.SemaphoreType.DMA,
      ],
  )
  def k(centers_hbm, tgt_hbm, emb_hbm, out_hbm, idx_v, rows_v, emb_v, acc_v,
        gsem0, gsem1, esem0, esem1):
    wid = lax.axis_index("s") * _NC + lax.axis_index("c")
    base = wid * rows_per_w
    gsems = (gsem0, gsem1)
    esems = (esem0, esem1)

    def issue(kk, slot):
      row = kk // 2
      col = (kk % 2) * _CHUNK
      g = pltpu.async_copy(centers_hbm.at[idx_v.at[row, pl.ds(col, _CHUNK)]],
                           rows_v.at[slot], gsems[slot])
      e = pltpu.async_copy(emb_hbm.at[pl.ds(base + kk * _CHUNK, _CHUNK)],
                           emb_v.at[slot], esems[slot])
      return (g, e)

    pltpu.sync_copy(tgt_hbm.at[pl.ds(wid * idx_rows, idx_rows)], idx_v)
    for b in range(2):
      issue(b, b)

    accs = tuple(jnp.zeros((_L,), jnp.float32) for _ in range(groups))

    def round_body(g, accs):
      for b in range(2):
        kk = 2 * g + b
        row = kk // 2
        col = (kk % 2) * _CHUNK
        pltpu.make_async_copy(centers_hbm.at[idx_v.at[row, pl.ds(col, _CHUNK)]],
                              rows_v.at[b], gsems[b]).wait()
        pltpu.make_async_copy(emb_hbm.at[pl.ds(base + kk * _CHUNK, _CHUNK)],
                              emb_v.at[b], esems[b]).wait()

        @pl.when(g < n_chunks // 2 - 1)
        def _():
          issue(kk + 2, b)

        slot = b

        def row_body(r, accs):
          new = []
          for j in range(groups):
            d = emb_v[slot, r, pl.ds(j * _L, _L)] - rows_v[slot, r, pl.ds(j * _L, _L)]
            new.append(accs[j] + d * d)
          return tuple(new)

        accs = plsc.parallel_loop(0, _CHUNK, carry=accs, unroll=1)(row_body)
      return accs

    accs = lax.fori_loop(0, n_chunks // 2, round_body, accs)

    total = accs[0]
    for j in range(1, groups):
      total = total + accs[j]
    acc_v[...] = total
    pltpu.sync_copy(acc_v, out_hbm.at[wid])

  return k(centers, target_2d, embedding)


def kernel(embedding, target, centers):
  batch, dim = embedding.shape
  partials = _partial_sums(
      centers, target.astype(jnp.int32), embedding, batch=batch, dim=dim
  )
  dist = jnp.sqrt(jnp.sum(partials))
  return (_LAMDA * 0.5 / batch) * dist


# speedup vs baseline: 1.0398x; 1.0009x over previous
import functools

import jax
import jax.numpy as jnp
from jax import lax
from jax.experimental import pallas as pl
from jax.experimental.pallas import tpu as pltpu
from jax.experimental.pallas import tpu_sc as plsc

_LAMDA = 0.5
_NC = 2
_NS = 16
_NW = _NC * _NS
_L = 16
_IDXROW = 128
_CHUNK = 64


@functools.partial(jax.jit, static_argnames=("batch", "dim"))
def _partial_sums(centers, target_i32, embedding, *, batch, dim):
  rows_per_w = batch // _NW
  idx_rows = rows_per_w // _IDXROW
  groups = dim // _L
  n_chunks = rows_per_w // _CHUNK
  target_2d = target_i32.reshape(batch // _IDXROW, _IDXROW)

  mesh = plsc.VectorSubcoreMesh(
      core_axis_name="c", subcore_axis_name="s", num_cores=_NC, num_subcores=_NS
  )

  @functools.partial(
      pl.kernel,
      out_type=jax.ShapeDtypeStruct((_NW, _L), jnp.float32),
      mesh=mesh,
      scratch_types=[
          pltpu.VMEM((idx_rows, _IDXROW), jnp.int32),
          pltpu.VMEM((2, _CHUNK, dim), jnp.float32),
          pltpu.VMEM((2, _CHUNK, dim), jnp.float32),
          pltpu.VMEM((_L,), jnp.float32),
          pltpu.SemaphoreType.DMA,
          pltpu.SemaphoreType.DMA,
          pltpu.SemaphoreType.DMA,
          pltpu.SemaphoreType.DMA,
      ],
  )
  def k(centers_hbm, tgt_hbm, emb_hbm, out_hbm, idx_v, rows_v, emb_v, acc_v,
        gsem0, gsem1, esem0, esem1):
    wid = lax.axis_index("s") * _NC + lax.axis_index("c")
    base = wid * rows_per_w
    gsems = (gsem0, gsem1)
    esems = (esem0, esem1)

    def issue(kk, slot):
      row = kk // 2
      col = (kk % 2) * _CHUNK
      g = pltpu.async_copy(centers_hbm.at[idx_v.at[row, pl.ds(col, _CHUNK)]],
                           rows_v.at[slot], gsems[slot])
      e = pltpu.async_copy(emb_hbm.at[pl.ds(base + kk * _CHUNK, _CHUNK)],
                           emb_v.at[slot], esems[slot])
      return (g, e)

    pltpu.sync_copy(tgt_hbm.at[pl.ds(wid * idx_rows, idx_rows)], idx_v)
    for b in range(2):
      issue(b, b)

    accs = tuple(jnp.zeros((_L,), jnp.float32) for _ in range(groups))

    def round_body(g, accs):
      for b in range(2):
        kk = 2 * g + b
        row = kk // 2
        col = (kk % 2) * _CHUNK
        pltpu.make_async_copy(centers_hbm.at[idx_v.at[row, pl.ds(col, _CHUNK)]],
                              rows_v.at[b], gsems[b]).wait()
        pltpu.make_async_copy(emb_hbm.at[pl.ds(base + kk * _CHUNK, _CHUNK)],
                              emb_v.at[b], esems[b]).wait()

        @pl.when(g < n_chunks // 2 - 1)
        def _():
          issue(kk + 2, b)

        slot = b

        def row_body(r, accs):
          new = []
          for j in range(groups):
            d = emb_v[slot, r, pl.ds(j * _L, _L)] - rows_v[slot, r, pl.ds(j * _L, _L)]
            new.append(accs[j] + d * d)
          return tuple(new)

        accs = plsc.parallel_loop(0, _CHUNK, carry=accs, unroll=2)(row_body)
      return accs

    accs = lax.fori_loop(0, n_chunks // 2, round_body, accs)

    total = accs[0]
    for j in range(1, groups):
      total = total + accs[j]
    acc_v[...] = total
    pltpu.sync_copy(acc_v, out_hbm.at[wid])

  return k(centers, target_2d, embedding)


def kernel(embedding, target, centers):
  batch, dim = embedding.shape
  partials = _partial_sums(
      centers, target.astype(jnp.int32), embedding, batch=batch, dim=dim
  )
  dist = jnp.sqrt(jnp.sum(partials))
  return (_LAMDA * 0.5 / batch) * dist
